# baseline (speedup 1.0000x reference)
import functools

import jax
import jax.numpy as jnp
from jax import lax
from jax.experimental import pallas as pl
from jax.experimental.pallas import tpu as pltpu
from jax.experimental.pallas import tpu_sc as plsc

N_NODES = 10000
D = 128
NC = 2
NS = 16
EDGE_B = 125
ROW_CH = 624
ROW_SPAN = 640
_ZCHUNK = 2000


def _make_deg(nb):
    mesh = plsc.VectorSubcoreMesh(core_axis_name="c", subcore_axis_name="s")

    @functools.partial(
        pl.kernel, mesh=mesh,
        out_type=jax.ShapeDtypeStruct((NC, N_NODES), jnp.float32),
        scratch_types=[
            pltpu.VMEM((nb, EDGE_B), jnp.int32),
            pltpu.VMEM((128,), jnp.float32),
            pltpu.VMEM((_ZCHUNK,), jnp.float32),
            pltpu.VMEM_SHARED((N_NODES,), jnp.float32),
        ],
    )
    def deg_k(dst_hbm, deg_hbm, dbuf, ones_v, zbuf, deg_sh):
        cid = lax.axis_index("c")
        sid = lax.axis_index("s")
        for k in range(128 // 16):
            ones_v[pl.ds(k * 16, 16)] = jnp.ones((16,), jnp.float32)

        def zb(i, c):
            zbuf[pl.ds(i * 16, 16)] = jnp.zeros((16,), jnp.float32)
            return c
        lax.fori_loop(0, _ZCHUNK // 16, zb, 0)

        @pl.when(sid == 0)
        def _():
            def zcopy(k, c):
                pltpu.sync_copy(zbuf, deg_sh.at[pl.ds(k * _ZCHUNK, _ZCHUNK)])
                return c
            lax.fori_loop(0, N_NODES // _ZCHUNK, zcopy, 0)

        plsc.subcore_barrier()
        pltpu.sync_copy(dst_hbm.at[cid, sid], dbuf)

        def body(j, c):
            pltpu.sync_copy(ones_v.at[pl.ds(0, EDGE_B)],
                            deg_sh.at[dbuf.at[j]], add=True)
            return c
        lax.fori_loop(0, nb, body, 0)

        plsc.subcore_barrier()

        @pl.when(sid == 0)
        def _():
            pltpu.sync_copy(deg_sh, deg_hbm.at[cid])

    return deg_k


def _make_agg(nb):
    mesh = plsc.VectorSubcoreMesh(core_axis_name="c", subcore_axis_name="s")

    assert nb % 2 == 0
    hb = nb // 2
    assert hb % 2 == 0

    @functools.partial(
        pl.kernel, mesh=mesh,
        out_type=jax.ShapeDtypeStruct((NC, N_NODES, D), jnp.float32),
        scratch_types=[
            pltpu.VMEM((hb, EDGE_B), jnp.int32),
            pltpu.VMEM((hb, EDGE_B), jnp.int32),
            pltpu.VMEM((EDGE_B, D), jnp.float32),
            pltpu.VMEM((EDGE_B, D), jnp.float32),
            pltpu.VMEM_SHARED((N_NODES, D), jnp.float32),
            pltpu.SemaphoreType.DMA,
            pltpu.SemaphoreType.DMA,
            pltpu.SemaphoreType.DMA,
            pltpu.SemaphoreType.DMA,
        ],
    )
    def agg_k(src_hbm, dst_hbm, u_hbm, acc_hbm,
              sbuf, dbuf, rows_a, rows_b, acc_sh,
              sem_a, sem_b, sem_sa, sem_sb):
        cid = lax.axis_index("c")
        sid = lax.axis_index("s")

        def wait_g(rows, sem):
            pltpu.make_async_copy(u_hbm.at[sbuf.at[0]], rows, sem).wait()

        def wait_s(rows, sem):
            pltpu.make_async_copy(rows, acc_sh.at[dbuf.at[0]], sem).wait()

        pltpu.sync_copy(src_hbm.at[cid, sid, pl.ds(0, hb)], sbuf)
        pltpu.sync_copy(dst_hbm.at[cid, sid, pl.ds(0, hb)], dbuf)
        pltpu.async_copy(u_hbm.at[sbuf.at[0]], rows_b, sem_b)

        def zr(i, c):
            for k in range(D // 16):
                rows_a[i, pl.ds(k * 16, 16)] = jnp.zeros((16,), jnp.float32)
            return c
        lax.fori_loop(0, EDGE_B, zr, 0)

        r0 = sid * ROW_CH
        for k in range(ROW_SPAN // 80):
            pltpu.sync_copy(rows_a.at[pl.ds(0, 80)],
                            acc_sh.at[pl.ds(r0 + k * 80, 80)])
        plsc.subcore_barrier()

        for h in range(2):
            @pl.when(jnp.bool_(h == 1))
            def _():
                pltpu.sync_copy(src_hbm.at[cid, sid, pl.ds(hb, hb)], sbuf)
                pltpu.sync_copy(dst_hbm.at[cid, sid, pl.ds(hb, hb)], dbuf)
                pltpu.async_copy(u_hbm.at[sbuf.at[0]], rows_b, sem_b)
            pltpu.async_copy(u_hbm.at[sbuf.at[1]], rows_a, sem_a)

            def body(j, c):
                last = j >= hb // 2 - 1
                wait_g(rows_b, sem_b)
                pltpu.async_copy(rows_b, acc_sh.at[dbuf.at[2 * j]],
                                 sem_sb, add=True)
                wait_g(rows_a, sem_a)
                pltpu.async_copy(rows_a, acc_sh.at[dbuf.at[2 * j + 1]],
                                 sem_sa, add=True)
                wait_s(rows_b, sem_sb)

                @pl.when(jnp.logical_not(last))
                def _():
                    pltpu.async_copy(u_hbm.at[sbuf.at[2 * j + 2]], rows_b, sem_b)
                wait_s(rows_a, sem_sa)

                @pl.when(jnp.logical_not(last))
                def _():
                    pltpu.async_copy(u_hbm.at[sbuf.at[2 * j + 3]], rows_a, sem_a)
                return c
            lax.fori_loop(0, hb // 2, body, 0)

        plsc.subcore_barrier()
        pltpu.sync_copy(acc_sh.at[pl.ds(r0, ROW_SPAN)],
                        acc_hbm.at[cid, pl.ds(r0, ROW_SPAN)])

    return agg_k


_BM = 1000


def _lin_body(x_ref, w_ref, dp_ref, u_ref):
    deg = dp_ref[0] + dp_ref[1] + 1.0
    dis = lax.rsqrt(deg)
    h = jnp.dot(x_ref[...], w_ref[...], preferred_element_type=jnp.float32)
    u_ref[...] = h * dis


def _fin_body(acc_ref, u_ref, dp_ref, b_ref, o_ref):
    dis = lax.rsqrt(dp_ref[0] + dp_ref[1] + 1.0)
    s = acc_ref[0] + acc_ref[1] + u_ref[...]
    o_ref[...] = s * dis + b_ref[...]


_lin = pl.pallas_call(
    _lin_body,
    grid=(N_NODES // _BM,),
    in_specs=[
        pl.BlockSpec((_BM, D), lambda i: (i, 0)),
        pl.BlockSpec((D, D), lambda i: (0, 0)),
        pl.BlockSpec((NC, _BM, 1), lambda i: (0, i, 0)),
    ],
    out_specs=pl.BlockSpec((_BM, D), lambda i: (i, 0)),
    out_shape=jax.ShapeDtypeStruct((N_NODES, D), jnp.float32),
)

_fin = pl.pallas_call(
    _fin_body,
    grid=(N_NODES // _BM,),
    in_specs=[
        pl.BlockSpec((NC, _BM, D), lambda i: (0, i, 0)),
        pl.BlockSpec((_BM, D), lambda i: (i, 0)),
        pl.BlockSpec((NC, _BM, 1), lambda i: (0, i, 0)),
        pl.BlockSpec((1, D), lambda i: (0, 0)),
    ],
    out_specs=pl.BlockSpec((_BM, D), lambda i: (i, 0)),
    out_shape=jax.ShapeDtypeStruct((N_NODES, D), jnp.float32),
)


def kernel(x, adj_t, W, b):
    E = adj_t.shape[1]
    assert E % (NC * NS * EDGE_B) == 0
    nb = E // (NC * NS * EDGE_B)
    src = adj_t[0].reshape(NC, NS, nb, EDGE_B)
    dst = adj_t[1].reshape(NC, NS, nb, EDGE_B)

    degp = _make_deg(nb)(dst)
    degp3 = degp[:, :, None]
    u = _lin(x, W, degp3)
    accs = _make_agg(nb)(src, dst, u)
    return _fin(accs, u, degp3, b.reshape(1, D))


# speedup vs baseline: 1.1766x; 1.1766x over previous
import functools

import jax
import jax.numpy as jnp
from jax import lax
from jax.experimental import pallas as pl
from jax.experimental.pallas import tpu as pltpu
from jax.experimental.pallas import tpu_sc as plsc

N_NODES = 10000
D = 128
NC = 2
NS = 16
EDGE_B = 125
ROW_CH = 624
ROW_SPAN = 640
_ZCHUNK = 2000


def _make_deg(nb):
    mesh = plsc.VectorSubcoreMesh(core_axis_name="c", subcore_axis_name="s")

    @functools.partial(
        pl.kernel, mesh=mesh,
        out_type=jax.ShapeDtypeStruct((NC, N_NODES), jnp.float32),
        scratch_types=[
            pltpu.VMEM((nb, EDGE_B), jnp.int32),
            pltpu.VMEM((128,), jnp.float32),
            pltpu.VMEM((_ZCHUNK,), jnp.float32),
            pltpu.VMEM_SHARED((N_NODES,), jnp.float32),
        ],
    )
    def deg_k(dst_hbm, deg_hbm, dbuf, ones_v, zbuf, deg_sh):
        cid = lax.axis_index("c")
        sid = lax.axis_index("s")
        for k in range(128 // 16):
            ones_v[pl.ds(k * 16, 16)] = jnp.ones((16,), jnp.float32)

        def zb(i, c):
            zbuf[pl.ds(i * 16, 16)] = jnp.zeros((16,), jnp.float32)
            return c
        lax.fori_loop(0, _ZCHUNK // 16, zb, 0)

        @pl.when(sid == 0)
        def _():
            def zcopy(k, c):
                pltpu.sync_copy(zbuf, deg_sh.at[pl.ds(k * _ZCHUNK, _ZCHUNK)])
                return c
            lax.fori_loop(0, N_NODES // _ZCHUNK, zcopy, 0)

        plsc.subcore_barrier()
        pltpu.sync_copy(dst_hbm.at[cid, sid], dbuf)

        def body(j, c):
            pltpu.sync_copy(ones_v.at[pl.ds(0, EDGE_B)],
                            deg_sh.at[dbuf.at[j]], add=True)
            return c
        lax.fori_loop(0, nb, body, 0)

        plsc.subcore_barrier()

        @pl.when(sid == 0)
        def _():
            pltpu.sync_copy(deg_sh, deg_hbm.at[cid])

    return deg_k


def _make_agg(nb):
    mesh = plsc.VectorSubcoreMesh(core_axis_name="c", subcore_axis_name="s")

    assert nb % 2 == 0
    hb = nb // 2
    assert hb % 2 == 0

    @functools.partial(
        pl.kernel, mesh=mesh,
        out_type=jax.ShapeDtypeStruct((NC, N_NODES, D), jnp.float32),
        scratch_types=[
            pltpu.VMEM((hb, EDGE_B), jnp.int32),
            pltpu.VMEM((hb, EDGE_B), jnp.int32),
            pltpu.VMEM((EDGE_B, D), jnp.float32),
            pltpu.VMEM((EDGE_B, D), jnp.float32),
            pltpu.VMEM_SHARED((N_NODES, D), jnp.float32),
            pltpu.SemaphoreType.DMA,
            pltpu.SemaphoreType.DMA,
        ],
    )
    def agg_k(src_hbm, dst_hbm, u_hbm, acc_hbm,
              sbuf, dbuf, rows_a, rows_b, acc_sh, sem_a, sem_b):
        cid = lax.axis_index("c")
        sid = lax.axis_index("s")

        def wait_g(rows, sem):
            pltpu.make_async_copy(u_hbm.at[sbuf.at[0]], rows, sem).wait()

        pltpu.sync_copy(src_hbm.at[cid, sid, pl.ds(0, hb)], sbuf)
        pltpu.sync_copy(dst_hbm.at[cid, sid, pl.ds(0, hb)], dbuf)
        pltpu.async_copy(u_hbm.at[sbuf.at[0]], rows_b, sem_b)

        def zr(i, c):
            for k in range(D // 16):
                rows_a[i, pl.ds(k * 16, 16)] = jnp.zeros((16,), jnp.float32)
            return c
        lax.fori_loop(0, EDGE_B, zr, 0)

        r0 = sid * ROW_CH
        for k in range(ROW_SPAN // 80):
            pltpu.sync_copy(rows_a.at[pl.ds(0, 80)],
                            acc_sh.at[pl.ds(r0 + k * 80, 80)])
        plsc.subcore_barrier()

        for h in range(2):
            @pl.when(jnp.bool_(h == 1))
            def _():
                pltpu.sync_copy(src_hbm.at[cid, sid, pl.ds(hb, hb)], sbuf)
                pltpu.sync_copy(dst_hbm.at[cid, sid, pl.ds(hb, hb)], dbuf)
                pltpu.async_copy(u_hbm.at[sbuf.at[0]], rows_b, sem_b)
            def body(j, c):
                pltpu.async_copy(u_hbm.at[sbuf.at[2 * j + 1]], rows_a, sem_a)
                wait_g(rows_b, sem_b)
                pltpu.sync_copy(rows_b, acc_sh.at[dbuf.at[2 * j]], add=True)

                @pl.when(j < hb // 2 - 1)
                def _():
                    pltpu.async_copy(u_hbm.at[sbuf.at[2 * j + 2]], rows_b, sem_b)
                wait_g(rows_a, sem_a)
                pltpu.sync_copy(rows_a, acc_sh.at[dbuf.at[2 * j + 1]], add=True)
                return c
            lax.fori_loop(0, hb // 2, body, 0)

        plsc.subcore_barrier()
        pltpu.sync_copy(acc_sh.at[pl.ds(r0, ROW_SPAN)],
                        acc_hbm.at[cid, pl.ds(r0, ROW_SPAN)])

    return agg_k


_BM = 1000


def _lin_body(x_ref, w_ref, dp_ref, u_ref):
    deg = dp_ref[0] + dp_ref[1] + 1.0
    dis = lax.rsqrt(deg)
    h = jnp.dot(x_ref[...], w_ref[...], preferred_element_type=jnp.float32)
    u_ref[...] = h * dis


def _fin_body(acc_ref, u_ref, dp_ref, b_ref, o_ref):
    dis = lax.rsqrt(dp_ref[0] + dp_ref[1] + 1.0)
    s = acc_ref[0] + acc_ref[1] + u_ref[...]
    o_ref[...] = s * dis + b_ref[...]


_lin = pl.pallas_call(
    _lin_body,
    grid=(N_NODES // _BM,),
    in_specs=[
        pl.BlockSpec((_BM, D), lambda i: (i, 0)),
        pl.BlockSpec((D, D), lambda i: (0, 0)),
        pl.BlockSpec((NC, _BM, 1), lambda i: (0, i, 0)),
    ],
    out_specs=pl.BlockSpec((_BM, D), lambda i: (i, 0)),
    out_shape=jax.ShapeDtypeStruct((N_NODES, D), jnp.float32),
)

_fin = pl.pallas_call(
    _fin_body,
    grid=(N_NODES // _BM,),
    in_specs=[
        pl.BlockSpec((NC, _BM, D), lambda i: (0, i, 0)),
        pl.BlockSpec((_BM, D), lambda i: (i, 0)),
        pl.BlockSpec((NC, _BM, 1), lambda i: (0, i, 0)),
        pl.BlockSpec((1, D), lambda i: (0, 0)),
    ],
    out_specs=pl.BlockSpec((_BM, D), lambda i: (i, 0)),
    out_shape=jax.ShapeDtypeStruct((N_NODES, D), jnp.float32),
)


def kernel(x, adj_t, W, b):
    E = adj_t.shape[1]
    assert E % (NC * NS * EDGE_B) == 0
    nb = E // (NC * NS * EDGE_B)
    src = adj_t[0].reshape(NC, NS, nb, EDGE_B)
    dst = adj_t[1].reshape(NC, NS, nb, EDGE_B)

    degp = _make_deg(nb)(dst)
    degp3 = degp[:, :, None]
    u = _lin(x, W, degp3)
    accs = _make_agg(nb)(src, dst, u)
    return _fin(accs, u, degp3, b.reshape(1, D))
